# baseline (speedup 1.0000x reference)
import functools

import jax
import jax.numpy as jnp
import numpy as np
from jax import lax
from jax.experimental import pallas as pl
from jax.experimental.pallas import tpu as pltpu
from jax.experimental.pallas import tpu_sc as plsc

BOHR = 0.529177
CUT_OFF = 20.0
CRIT0 = 0.63
CRIT1 = 1.26

_N = 10000
_NPAD = 10240
_BN = 256
_D = 256
_H = 128
_GS = 16
_DH = 2048
_E = 160000
_NW = 32
_EPW = 5120
_EPAD = _NW * _EPW
_UNROLL = 4


def _mlp_body(a_ref, w1_ref, aux_ref, oh_ref, out_ref):
    a = a_ref[...]
    w1 = w1_ref[...]
    b1 = aux_ref[0:1, :]
    w2 = aux_ref[1:2, :]
    h = jnp.tanh(jnp.dot(a, w1, preferred_element_type=jnp.float32) + b1)
    hw = h * w2
    cols = [jnp.sum(hw[:, j * _H:(j + 1) * _H], axis=1, keepdims=True)
            for j in range(_GS)]
    out16 = jnp.concatenate(cols, axis=1)
    out16 = out16 + aux_ref[2:3, 0:_GS]
    oh = oh_ref[...]
    sel = out16 * oh
    colg = lax.broadcasted_iota(jnp.int32, (1, _GS), 1) // 4
    mg = []
    for g in range(4):
        mg.append(jnp.sum(jnp.where(colg == g, sel, 0.0), axis=1, keepdims=True))

    def softplus(x):
        return jnp.maximum(x, 0.0) + jnp.log(1.0 + jnp.exp(-jnp.abs(x)))

    m1 = softplus(mg[0]) + 1e-3
    m2 = softplus(mg[1]) + 1e-3
    m3 = softplus(mg[2]) + 1e-3
    v = softplus(mg[3]) + 1e-3
    ratio = aux_ref[3:4, 0:_GS]
    rsel = jnp.sum(jnp.where(colg == 0, oh * ratio, 0.0), axis=1, keepdims=True)
    polar = rsel * v
    out_ref[...] = jnp.concatenate([m1, m2, m3, polar], axis=1)


def _mlp_stage(aev_p, w1r, aux, oh_p):
    return pl.pallas_call(
        _mlp_body,
        grid=(_NPAD // _BN,),
        in_specs=[
            pl.BlockSpec((_BN, _D), lambda i: (i, 0)),
            pl.BlockSpec((_D, _DH), lambda i: (0, 0)),
            pl.BlockSpec((8, _DH), lambda i: (0, 0)),
            pl.BlockSpec((_BN, _GS), lambda i: (i, 0)),
        ],
        out_specs=pl.BlockSpec((_BN, 4), lambda i: (i, 0)),
        out_shape=jax.ShapeDtypeStruct((_NPAD, 4), jnp.float32),
    )(aev_p, w1r, aux, oh_p)


def _sqrt16(x):
    b = lax.bitcast_convert_type(x, jnp.int32)
    y = lax.bitcast_convert_type((b >> 1) + jnp.int32(0x1FBD1DF5), jnp.float32)
    y = 0.5 * (y + x / y)
    y = 0.5 * (y + x / y)
    y = 0.5 * (y + x / y)
    return y


def _pair_energy(m1s, m2s, m3s, ps, xs, ys, zs,
                 m1d, m2d, m3d, pd, xd, yd, zd):
    dx = xd - xs
    dy = yd - ys
    dz = zd - zs
    r = dx * dx + dy * dy + dz * dz + 1e-12
    scaled = m1s / ps + m1d / pd
    c6 = m1s * m1d / scaled
    c8 = 1.5 * (m1s * m2d + m2s * m1d) / scaled
    c10 = 2.0 * (m1s * m3d + m3s * m1d + 2.1 * m2s * m2d) / scaled
    rcrit = (_sqrt16(c8 / c6) + _sqrt16(_sqrt16(c10 / c6))
             + _sqrt16(c10 / c8)) * (1.0 / 3.0)
    rvdw = CRIT0 + CRIT1 * BOHR * rcrit
    rv2 = rvdw * rvdw
    rv6 = rv2 * rv2 * rv2
    rv10 = rv6 * rv2 * rv2
    rc2 = CUT_OFF * CUT_OFF
    ro = 0.66 * 0.66 * rc2
    cut = jnp.where(
        r < ro, 1.0,
        (rc2 - r) * (rc2 - r) * (rc2 + 2.0 * r - 3.0 * ro) * (1.0 / (rc2 - ro) ** 3))
    r3 = r * r * r
    r4 = r3 * r
    r5 = r4 * r
    b2 = BOHR * BOHR
    b6 = b2 * b2 * b2
    b8 = b6 * b2
    b10 = b8 * b2
    e = -(c6 / (r3 + rv6) * b6 + c8 / (r4 + rv6) * b8
          + c10 / (r5 + rv10) * b10) * cut
    return e


def _edge_body(f0, f1, f2, f3, f4, f5, f6, sidx, didx, out,
               sidx_v, didx_v, sfb, dfb, acc, sem):
    feat = [f0, f1, f2, f3, f4, f5, f6]
    wid = lax.axis_index("s") * 2 + lax.axis_index("c")
    pltpu.sync_copy(sidx.at[pl.ds(wid * _EPW, _EPW)], sidx_v)
    pltpu.sync_copy(didx.at[pl.ds(wid * _EPW, _EPW)], didx_v)
    acc[...] = jnp.zeros((16,), jnp.float32)
    lane = lax.iota(jnp.int32, 16)
    ebase0 = wid * _EPW

    cps = []
    for f in range(7):
        cps.append(pltpu.async_copy(feat[f].at[sidx_v], sfb[f], sem))
        cps.append(pltpu.async_copy(feat[f].at[didx_v], dfb[f], sem))
    for cp in cps:
        cp.wait()

    def group(i, carry):
        a = acc[...]
        for u in range(_UNROLL):
            off = i * (16 * _UNROLL) + u * 16
            sl = pl.ds(off, 16)
            fs = [sfb[f][sl] for f in range(7)]
            fd = [dfb[f][sl] for f in range(7)]
            e = _pair_energy(*fs, *fd)
            eid = ebase0 + off + lane
            a = a + jnp.where(eid < _E, e, 0.0)
        acc[...] = a
        return carry

    lax.fori_loop(0, _EPW // (16 * _UNROLL), group, 0)
    pltpu.sync_copy(acc, out.at[wid])


def _edge_stage(feats, sidxp, didxp):
    mesh = plsc.VectorSubcoreMesh(core_axis_name="c", subcore_axis_name="s")
    fn = functools.partial(
        pl.kernel,
        mesh=mesh,
        out_type=jax.ShapeDtypeStruct((_NW, 16), jnp.float32),
        scratch_types=[
            pltpu.VMEM((_EPW,), jnp.int32),
            pltpu.VMEM((_EPW,), jnp.int32),
            [pltpu.VMEM((_EPW,), jnp.float32) for _ in range(7)],
            [pltpu.VMEM((_EPW,), jnp.float32) for _ in range(7)],
            pltpu.VMEM((16,), jnp.float32),
            pltpu.SemaphoreType.DMA,
        ],
    )(_edge_body)
    return fn(*feats, sidxp, didxp)


def kernel(atomic_index, aev, positions, edge_index, W1, b1, W2, b2,
           v_free, polar_free):
    n, d = aev.shape
    assert n == _N and d == _D
    aev_p = jnp.pad(aev, ((0, _NPAD - _N), (0, 0)))
    oh = (atomic_index[:, None] == (jnp.arange(_GS, dtype=jnp.int32)[None, :] % 4)
          ).astype(jnp.float32)
    oh_p = jnp.pad(oh, ((0, _NPAD - _N), (0, 0)))
    w1r = jnp.transpose(W1, (2, 0, 1, 3)).reshape(_D, _DH)
    ratio = polar_free / v_free
    aux = (jnp.zeros((8, _DH), jnp.float32)
           .at[0].set(b1.reshape(_DH))
           .at[1].set(W2.reshape(_DH))
           .at[2, 0:_GS].set(b2.reshape(_GS))
           .at[3, 0:_GS].set(jnp.tile(ratio, 4)))
    m4 = _mlp_stage(aev_p, w1r, aux, oh_p)
    pos_p = jnp.pad(positions, ((0, _NPAD - _N), (0, 0)))
    feats = [m4[:, 0], m4[:, 1], m4[:, 2], m4[:, 3],
             pos_p[:, 0], pos_p[:, 1], pos_p[:, 2]]
    sidxp = jnp.pad(edge_index[0], (0, _EPAD - _E))
    didxp = jnp.pad(edge_index[1], (0, _EPAD - _E))
    parts = _edge_stage(feats, sidxp, didxp)
    return jnp.sum(parts)


# speedup vs baseline: 1.6822x; 1.6822x over previous
import functools

import jax
import jax.numpy as jnp
import numpy as np
from jax import lax
from jax.experimental import pallas as pl
from jax.experimental.pallas import tpu as pltpu
from jax.experimental.pallas import tpu_sc as plsc

BOHR = 0.529177
CUT_OFF = 20.0
CRIT0 = 0.63
CRIT1 = 1.26

_N = 10000
_NPAD = 10240
_BN = 256
_D = 256
_H = 128
_GS = 16
_DH = 2048
_E = 160000
_NW = 32
_CW = 128
_CHUNKS = 40
_EPW = _CHUNKS * _CW
_EPAD = _NW * _EPW


def _mlp_body(a_ref, w1_ref, aux_ref, oh_ref, out_ref):
    a = a_ref[...]
    w1 = w1_ref[...]
    b1 = aux_ref[0:1, :]
    w2 = aux_ref[1:2, :]
    h = jnp.tanh(jnp.dot(a, w1, preferred_element_type=jnp.float32) + b1)
    hw = h * w2
    cols = [jnp.sum(hw[:, j * _H:(j + 1) * _H], axis=1, keepdims=True)
            for j in range(_GS)]
    out16 = jnp.concatenate(cols, axis=1)
    out16 = out16 + aux_ref[2:3, 0:_GS]
    oh = oh_ref[...]
    sel = out16 * oh
    colg = lax.broadcasted_iota(jnp.int32, (1, _GS), 1) // 4
    mg = []
    for g in range(4):
        mg.append(jnp.sum(jnp.where(colg == g, sel, 0.0), axis=1, keepdims=True))

    def softplus(x):
        return jnp.maximum(x, 0.0) + jnp.log(1.0 + jnp.exp(-jnp.abs(x)))

    m1 = softplus(mg[0]) + 1e-3
    m2 = softplus(mg[1]) + 1e-3
    m3 = softplus(mg[2]) + 1e-3
    v = softplus(mg[3]) + 1e-3
    ratio = aux_ref[3:4, 0:_GS]
    rsel = jnp.sum(jnp.where(colg == 0, oh * ratio, 0.0), axis=1, keepdims=True)
    polar = rsel * v
    out_ref[...] = jnp.concatenate([m1, m2, m3, polar], axis=1)


def _mlp_stage(aev_p, w1r, aux, oh_p):
    return pl.pallas_call(
        _mlp_body,
        grid=(_NPAD // _BN,),
        in_specs=[
            pl.BlockSpec((_BN, _D), lambda i: (i, 0)),
            pl.BlockSpec((_D, _DH), lambda i: (0, 0)),
            pl.BlockSpec((8, _DH), lambda i: (0, 0)),
            pl.BlockSpec((_BN, _GS), lambda i: (i, 0)),
        ],
        out_specs=pl.BlockSpec((_BN, 4), lambda i: (i, 0)),
        out_shape=jax.ShapeDtypeStruct((_NPAD, 4), jnp.float32),
    )(aev_p, w1r, aux, oh_p)


def _sqrt16(x):
    b = lax.bitcast_convert_type(x, jnp.int32)
    y = lax.bitcast_convert_type((b >> 1) + jnp.int32(0x1FBD1DF5), jnp.float32)
    y = 0.5 * (y + x / y)
    y = 0.5 * (y + x / y)
    y = 0.5 * (y + x / y)
    return y


def _pair_energy(m1s, m2s, m3s, ps, xs, ys, zs,
                 m1d, m2d, m3d, pd, xd, yd, zd):
    dx = xd - xs
    dy = yd - ys
    dz = zd - zs
    r = dx * dx + dy * dy + dz * dz + 1e-12
    scaled = m1s / ps + m1d / pd
    c6 = m1s * m1d / scaled
    c8 = 1.5 * (m1s * m2d + m2s * m1d) / scaled
    c10 = 2.0 * (m1s * m3d + m3s * m1d + 2.1 * m2s * m2d) / scaled
    rcrit = (_sqrt16(c8 / c6) + _sqrt16(_sqrt16(c10 / c6))
             + _sqrt16(c10 / c8)) * (1.0 / 3.0)
    rvdw = CRIT0 + CRIT1 * BOHR * rcrit
    rv2 = rvdw * rvdw
    rv6 = rv2 * rv2 * rv2
    rv10 = rv6 * rv2 * rv2
    rc2 = CUT_OFF * CUT_OFF
    ro = 0.66 * 0.66 * rc2
    cut = jnp.where(
        r < ro, 1.0,
        (rc2 - r) * (rc2 - r) * (rc2 + 2.0 * r - 3.0 * ro) * (1.0 / (rc2 - ro) ** 3))
    r3 = r * r * r
    r4 = r3 * r
    r5 = r4 * r
    b2 = BOHR * BOHR
    b6 = b2 * b2 * b2
    b8 = b6 * b2
    b10 = b8 * b2
    e = -(c6 / (r3 + rv6) * b6 + c8 / (r4 + rv6) * b8
          + c10 / (r5 + rv10) * b10) * cut
    return e


def _edge_body(f0, f1, f2, f3, f4, f5, f6, sidx, didx, out,
               sidx_v, didx_v, sfb, dfb, acc, sem0, sem1):
    feat = [f0, f1, f2, f3, f4, f5, f6]
    sems = [sem0, sem1]
    wid = lax.axis_index("s") * 2 + lax.axis_index("c")
    base_row = wid * _CHUNKS
    pltpu.sync_copy(sidx.at[pl.ds(base_row, _CHUNKS)], sidx_v)
    pltpu.sync_copy(didx.at[pl.ds(base_row, _CHUNKS)], didx_v)
    acc[...] = jnp.zeros((16,), jnp.float32)
    lane = lax.iota(jnp.int32, 16)
    ebase0 = wid * _EPW

    def fire(j, b):
        for f in range(7):
            pltpu.async_copy(feat[f].at[sidx_v.at[j]], sfb[f].at[b], sems[b])
            pltpu.async_copy(feat[f].at[didx_v.at[j]], dfb[f].at[b], sems[b])

    def drain(b):
        for f in range(7):
            pltpu.make_async_copy(
                feat[f].at[pl.ds(0, _CW)], sfb[f].at[b], sems[b]).wait()
            pltpu.make_async_copy(
                feat[f].at[pl.ds(0, _CW)], dfb[f].at[b], sems[b]).wait()

    fire(0, 0)
    fire(1, 1)

    def outer(i, carry):
        for b in range(2):
            j = 2 * i + b
            drain(b)
            a = acc[...]
            for u in range(_CW // 16):
                sl = pl.ds(u * 16, 16)
                fs = [sfb[f][b, sl] for f in range(7)]
                fd = [dfb[f][b, sl] for f in range(7)]
                e = _pair_energy(*fs, *fd)
                eid = ebase0 + j * _CW + u * 16 + lane
                a = a + jnp.where(eid < _E, e, 0.0)
            acc[...] = a

            @pl.when(j + 2 < _CHUNKS)
            def _():
                fire(j + 2, b)
        return carry

    lax.fori_loop(0, _CHUNKS // 2, outer, 0)
    pltpu.sync_copy(acc, out.at[wid])


def _edge_stage(feats, sidxp, didxp):
    mesh = plsc.VectorSubcoreMesh(core_axis_name="c", subcore_axis_name="s")
    fn = functools.partial(
        pl.kernel,
        mesh=mesh,
        out_type=jax.ShapeDtypeStruct((_NW, 16), jnp.float32),
        scratch_types=[
            pltpu.VMEM((_CHUNKS, _CW), jnp.int32),
            pltpu.VMEM((_CHUNKS, _CW), jnp.int32),
            [pltpu.VMEM((2, _CW), jnp.float32) for _ in range(7)],
            [pltpu.VMEM((2, _CW), jnp.float32) for _ in range(7)],
            pltpu.VMEM((16,), jnp.float32),
            pltpu.SemaphoreType.DMA,
            pltpu.SemaphoreType.DMA,
        ],
    )(_edge_body)
    return fn(*feats, sidxp, didxp)


def kernel(atomic_index, aev, positions, edge_index, W1, b1, W2, b2,
           v_free, polar_free):
    n, d = aev.shape
    assert n == _N and d == _D
    aev_p = jnp.pad(aev, ((0, _NPAD - _N), (0, 0)))
    oh = (atomic_index[:, None] == (jnp.arange(_GS, dtype=jnp.int32)[None, :] % 4)
          ).astype(jnp.float32)
    oh_p = jnp.pad(oh, ((0, _NPAD - _N), (0, 0)))
    w1r = jnp.transpose(W1, (2, 0, 1, 3)).reshape(_D, _DH)
    ratio = polar_free / v_free
    aux = (jnp.zeros((8, _DH), jnp.float32)
           .at[0].set(b1.reshape(_DH))
           .at[1].set(W2.reshape(_DH))
           .at[2, 0:_GS].set(b2.reshape(_GS))
           .at[3, 0:_GS].set(jnp.tile(ratio, 4)))
    m4 = _mlp_stage(aev_p, w1r, aux, oh_p)
    pos_p = jnp.pad(positions, ((0, _NPAD - _N), (0, 0)))
    feats = [m4[:, 0], m4[:, 1], m4[:, 2], m4[:, 3],
             pos_p[:, 0], pos_p[:, 1], pos_p[:, 2]]
    sidxp = jnp.pad(edge_index[0], (0, _EPAD - _E)).reshape(
        _NW * _CHUNKS, _CW)
    didxp = jnp.pad(edge_index[1], (0, _EPAD - _E)).reshape(
        _NW * _CHUNKS, _CW)
    parts = _edge_stage(feats, sidxp, didxp)
    return jnp.sum(parts)


# speedup vs baseline: 2.3607x; 1.4033x over previous
import functools

import jax
import jax.numpy as jnp
import numpy as np
from jax import lax
from jax.experimental import pallas as pl
from jax.experimental.pallas import tpu as pltpu
from jax.experimental.pallas import tpu_sc as plsc

BOHR = 0.529177
CUT_OFF = 20.0
CRIT0 = 0.63
CRIT1 = 1.26

_N = 10000
_NPAD = 10240
_BN = 256
_D = 256
_H = 128
_GS = 16
_DH = 2048
_E = 160000
_NW = 32
_CW = 128
_CHUNKS = 40
_EPW = _CHUNKS * _CW
_EPAD = _NW * _EPW


def _mlp_body(a_ref, w1_ref, aux_ref, oh_ref, out_ref):
    a = a_ref[...]
    w1 = w1_ref[...]
    b1 = aux_ref[0:1, :]
    w2 = aux_ref[1:2, :]
    h = jnp.tanh(jnp.dot(a, w1, preferred_element_type=jnp.float32) + b1)
    hw = h * w2
    cols = [jnp.sum(hw[:, j * _H:(j + 1) * _H], axis=1, keepdims=True)
            for j in range(_GS)]
    out16 = jnp.concatenate(cols, axis=1)
    out16 = out16 + aux_ref[2:3, 0:_GS]
    oh = oh_ref[...]
    sel = out16 * oh
    colg = lax.broadcasted_iota(jnp.int32, (1, _GS), 1) // 4
    mg = []
    for g in range(4):
        mg.append(jnp.sum(jnp.where(colg == g, sel, 0.0), axis=1, keepdims=True))

    def softplus(x):
        return jnp.maximum(x, 0.0) + jnp.log(1.0 + jnp.exp(-jnp.abs(x)))

    m1 = softplus(mg[0]) + 1e-3
    m2 = softplus(mg[1]) + 1e-3
    m3 = softplus(mg[2]) + 1e-3
    v = softplus(mg[3]) + 1e-3
    ratio = aux_ref[3:4, 0:_GS]
    rsel = jnp.sum(jnp.where(colg == 0, oh * ratio, 0.0), axis=1, keepdims=True)
    polar = rsel * v
    out_ref[...] = jnp.concatenate([m1, m2, m3, polar], axis=1)


def _mlp_stage(aev_p, w1r, aux, oh_p):
    return pl.pallas_call(
        _mlp_body,
        grid=(_NPAD // _BN,),
        in_specs=[
            pl.BlockSpec((_BN, _D), lambda i: (i, 0)),
            pl.BlockSpec((_D, _DH), lambda i: (0, 0)),
            pl.BlockSpec((8, _DH), lambda i: (0, 0)),
            pl.BlockSpec((_BN, _GS), lambda i: (i, 0)),
        ],
        out_specs=pl.BlockSpec((_BN, 4), lambda i: (i, 0)),
        out_shape=jax.ShapeDtypeStruct((_NPAD, 4), jnp.float32),
    )(aev_p, w1r, aux, oh_p)


def _sqrt16(x):
    b = lax.bitcast_convert_type(x, jnp.int32)
    y = lax.bitcast_convert_type((b >> 1) + jnp.int32(0x1FBD1DF5), jnp.float32)
    y = 0.5 * (y + x / y)
    y = 0.5 * (y + x / y)
    y = 0.5 * (y + x / y)
    return y


def _transpose16(rows, nout=7):
    lane = lax.iota(jnp.int32, 16)
    stages = (8, 4, 2, 1)
    needed = [set(range(nout))]
    for o in reversed(stages):
        needed.append({f ^ b for f in needed[-1] for b in (0, o)})
    needed = needed[::-1]
    cur = {i: rows[i] for i in needed[0]}
    for s, o in enumerate(stages):
        mask = ((lane // o) % 2) == 0
        idx_r = (lane - o) & 15
        idx_l = (lane + o) & 15
        new = {}
        for i in needed[s + 1]:
            j = i ^ o
            if i < j:
                rot = cur[j].at[idx_r].get(mode="promise_in_bounds")
                new[i] = jnp.where(mask, cur[i], rot)
            else:
                rot = cur[j].at[idx_l].get(mode="promise_in_bounds")
                new[i] = jnp.where(mask, rot, cur[i])
        cur = new
    return [cur[f] for f in range(nout)]


def _pair_energy(m1s, m2s, m3s, ps, xs, ys, zs,
                 m1d, m2d, m3d, pd, xd, yd, zd):
    dx = xd - xs
    dy = yd - ys
    dz = zd - zs
    r = dx * dx + dy * dy + dz * dz + 1e-12
    scaled = m1s / ps + m1d / pd
    c6 = m1s * m1d / scaled
    c8 = 1.5 * (m1s * m2d + m2s * m1d) / scaled
    c10 = 2.0 * (m1s * m3d + m3s * m1d + 2.1 * m2s * m2d) / scaled
    rcrit = (_sqrt16(c8 / c6) + _sqrt16(_sqrt16(c10 / c6))
             + _sqrt16(c10 / c8)) * (1.0 / 3.0)
    rvdw = CRIT0 + CRIT1 * BOHR * rcrit
    rv2 = rvdw * rvdw
    rv6 = rv2 * rv2 * rv2
    rv10 = rv6 * rv2 * rv2
    rc2 = CUT_OFF * CUT_OFF
    ro = 0.66 * 0.66 * rc2
    cut = jnp.where(
        r < ro, 1.0,
        (rc2 - r) * (rc2 - r) * (rc2 + 2.0 * r - 3.0 * ro) * (1.0 / (rc2 - ro) ** 3))
    r3 = r * r * r
    r4 = r3 * r
    r5 = r4 * r
    b2 = BOHR * BOHR
    b6 = b2 * b2 * b2
    b8 = b6 * b2
    b10 = b8 * b2
    e = -(c6 / (r3 + rv6) * b6 + c8 / (r4 + rv6) * b8
          + c10 / (r5 + rv10) * b10) * cut
    return e


def _edge_body(f0, f1, f2, f3, f4, f5, f6, sidx, didx, out,
               sh, sidx_v, didx_v, sfb, dfb, acc, sem0, sem1):
    feat = [f0, f1, f2, f3, f4, f5, f6]
    sems = [sem0, sem1]
    sid = lax.axis_index("s")
    wid = sid * 2 + lax.axis_index("c")

    @pl.when(sid == 0)
    def _():
        for f in range(7):
            pltpu.sync_copy(feat[f], sh[f])
    plsc.subcore_barrier()

    base_row = wid * _CHUNKS
    pltpu.sync_copy(sidx.at[pl.ds(base_row, _CHUNKS)], sidx_v)
    pltpu.sync_copy(didx.at[pl.ds(base_row, _CHUNKS)], didx_v)
    acc[...] = jnp.zeros((16,), jnp.float32)
    lane = lax.iota(jnp.int32, 16)
    ebase0 = wid * _EPW

    def fire(j, b):
        for f in range(7):
            pltpu.async_copy(sh[f].at[sidx_v.at[j]], sfb[f].at[b], sems[b])
            pltpu.async_copy(sh[f].at[didx_v.at[j]], dfb[f].at[b], sems[b])

    def drain(b):
        for f in range(7):
            pltpu.make_async_copy(
                sh[f].at[pl.ds(0, _CW)], sfb[f].at[b], sems[b]).wait()
            pltpu.make_async_copy(
                sh[f].at[pl.ds(0, _CW)], dfb[f].at[b], sems[b]).wait()

    fire(0, 0)
    fire(1, 1)

    def outer(i, carry):
        for b in range(2):
            j = 2 * i + b
            drain(b)
            a = acc[...]
            for u in range(_CW // 16):
                sl = pl.ds(u * 16, 16)
                fs = [sfb[f][b, sl] for f in range(7)]
                fd = [dfb[f][b, sl] for f in range(7)]
                e = _pair_energy(*fs, *fd)
                eid = ebase0 + j * _CW + u * 16 + lane
                a = a + jnp.where(eid < _E, e, 0.0)
            acc[...] = a

            @pl.when(j + 2 < _CHUNKS)
            def _():
                fire(j + 2, b)
        return carry

    lax.fori_loop(0, _CHUNKS // 2, outer, 0)
    pltpu.sync_copy(acc, out.at[wid])


def _edge_stage(feats, sidxp, didxp):
    mesh = plsc.VectorSubcoreMesh(core_axis_name="c", subcore_axis_name="s")
    fn = functools.partial(
        pl.kernel,
        mesh=mesh,
        out_type=jax.ShapeDtypeStruct((_NW, 16), jnp.float32),
        scratch_types=[
            [pltpu.VMEM_SHARED((_NPAD,), jnp.float32) for _ in range(7)],
            pltpu.VMEM((_CHUNKS, _CW), jnp.int32),
            pltpu.VMEM((_CHUNKS, _CW), jnp.int32),
            [pltpu.VMEM((2, _CW), jnp.float32) for _ in range(7)],
            [pltpu.VMEM((2, _CW), jnp.float32) for _ in range(7)],
            pltpu.VMEM((16,), jnp.float32),
            pltpu.SemaphoreType.DMA,
            pltpu.SemaphoreType.DMA,
        ],
    )(_edge_body)
    return fn(*feats, sidxp, didxp)


def kernel(atomic_index, aev, positions, edge_index, W1, b1, W2, b2,
           v_free, polar_free):
    n, d = aev.shape
    assert n == _N and d == _D
    aev_p = jnp.pad(aev, ((0, _NPAD - _N), (0, 0)))
    oh = (atomic_index[:, None] == (jnp.arange(_GS, dtype=jnp.int32)[None, :] % 4)
          ).astype(jnp.float32)
    oh_p = jnp.pad(oh, ((0, _NPAD - _N), (0, 0)))
    w1r = jnp.transpose(W1, (2, 0, 1, 3)).reshape(_D, _DH)
    ratio = polar_free / v_free
    aux = (jnp.zeros((8, _DH), jnp.float32)
           .at[0].set(b1.reshape(_DH))
           .at[1].set(W2.reshape(_DH))
           .at[2, 0:_GS].set(b2.reshape(_GS))
           .at[3, 0:_GS].set(jnp.tile(ratio, 4)))
    m4 = _mlp_stage(aev_p, w1r, aux, oh_p)
    pos_p = jnp.pad(positions, ((0, _NPAD - _N), (0, 0)))
    feats = [m4[:, 0], m4[:, 1], m4[:, 2], m4[:, 3],
             pos_p[:, 0], pos_p[:, 1], pos_p[:, 2]]
    sidxp = jnp.pad(edge_index[0], (0, _EPAD - _E)).reshape(
        _NW * _CHUNKS, _CW)
    didxp = jnp.pad(edge_index[1], (0, _EPAD - _E)).reshape(
        _NW * _CHUNKS, _CW)
    parts = _edge_stage(feats, sidxp, didxp)
    return jnp.sum(parts)


# speedup vs baseline: 2.3648x; 1.0017x over previous
import functools

import jax
import jax.numpy as jnp
import numpy as np
from jax import lax
from jax.experimental import pallas as pl
from jax.experimental.pallas import tpu as pltpu
from jax.experimental.pallas import tpu_sc as plsc

BOHR = 0.529177
CUT_OFF = 20.0
CRIT0 = 0.63
CRIT1 = 1.26

_N = 10000
_NPAD = 10240
_BN = 256
_D = 256
_H = 128
_GS = 16
_DH = 2048
_E = 160000
_NW = 32
_CW = 128
_CHUNKS = 40
_EPW = _CHUNKS * _CW
_EPAD = _NW * _EPW


def _mlp_body(a_ref, w1_ref, aux_ref, oh_ref, out_ref):
    a = a_ref[...]
    w1 = w1_ref[...]
    b1 = aux_ref[0:1, :]
    w2 = aux_ref[1:2, :]
    h = jnp.tanh(jnp.dot(a, w1, preferred_element_type=jnp.float32)
                 .astype(jnp.float32) + b1)
    hw = h * w2
    cols = [jnp.sum(hw[:, j * _H:(j + 1) * _H], axis=1, keepdims=True)
            for j in range(_GS)]
    out16 = jnp.concatenate(cols, axis=1)
    out16 = out16 + aux_ref[2:3, 0:_GS]
    oh = oh_ref[...]
    sel = out16 * oh
    colg = lax.broadcasted_iota(jnp.int32, (1, _GS), 1) // 4
    mg = []
    for g in range(4):
        mg.append(jnp.sum(jnp.where(colg == g, sel, 0.0), axis=1, keepdims=True))

    def softplus(x):
        return jnp.maximum(x, 0.0) + jnp.log(1.0 + jnp.exp(-jnp.abs(x)))

    m1 = softplus(mg[0]) + 1e-3
    m2 = softplus(mg[1]) + 1e-3
    m3 = softplus(mg[2]) + 1e-3
    v = softplus(mg[3]) + 1e-3
    ratio = aux_ref[3:4, 0:_GS]
    rsel = jnp.sum(jnp.where(colg == 0, oh * ratio, 0.0), axis=1, keepdims=True)
    polar = rsel * v
    out_ref[...] = jnp.concatenate([m1, m2, m3, polar], axis=1)


def _mlp_stage(aev_p, w1r, aux, oh_p):
    return pl.pallas_call(
        _mlp_body,
        grid=(_NPAD // _BN,),
        in_specs=[
            pl.BlockSpec((_BN, _D), lambda i: (i, 0)),
            pl.BlockSpec((_D, _DH), lambda i: (0, 0)),
            pl.BlockSpec((8, _DH), lambda i: (0, 0)),
            pl.BlockSpec((_BN, _GS), lambda i: (i, 0)),
        ],
        out_specs=pl.BlockSpec((_BN, 4), lambda i: (i, 0)),
        out_shape=jax.ShapeDtypeStruct((_NPAD, 4), jnp.float32),
    )(aev_p, w1r, aux, oh_p)


def _sqrt16(x):
    b = lax.bitcast_convert_type(x, jnp.int32)
    y = lax.bitcast_convert_type((b >> 1) + jnp.int32(0x1FBD1DF5), jnp.float32)
    y = 0.5 * (y + x / y)
    y = 0.5 * (y + x / y)
    y = 0.5 * (y + x / y)
    return y


def _transpose16(rows, nout=7):
    lane = lax.iota(jnp.int32, 16)
    stages = (8, 4, 2, 1)
    needed = [set(range(nout))]
    for o in reversed(stages):
        needed.append({f ^ b for f in needed[-1] for b in (0, o)})
    needed = needed[::-1]
    cur = {i: rows[i] for i in needed[0]}
    for s, o in enumerate(stages):
        mask = ((lane // o) % 2) == 0
        idx_r = (lane - o) & 15
        idx_l = (lane + o) & 15
        new = {}
        for i in needed[s + 1]:
            j = i ^ o
            if i < j:
                rot = cur[j].at[idx_r].get(mode="promise_in_bounds")
                new[i] = jnp.where(mask, cur[i], rot)
            else:
                rot = cur[j].at[idx_l].get(mode="promise_in_bounds")
                new[i] = jnp.where(mask, rot, cur[i])
        cur = new
    return [cur[f] for f in range(nout)]


def _pair_energy(m1s, m2s, m3s, ps, xs, ys, zs,
                 m1d, m2d, m3d, pd, xd, yd, zd):
    dx = xd - xs
    dy = yd - ys
    dz = zd - zs
    r = dx * dx + dy * dy + dz * dz + 1e-12
    scaled = m1s / ps + m1d / pd
    c6 = m1s * m1d / scaled
    c8 = 1.5 * (m1s * m2d + m2s * m1d) / scaled
    c10 = 2.0 * (m1s * m3d + m3s * m1d + 2.1 * m2s * m2d) / scaled
    rcrit = (_sqrt16(c8 / c6) + _sqrt16(_sqrt16(c10 / c6))
             + _sqrt16(c10 / c8)) * (1.0 / 3.0)
    rvdw = CRIT0 + CRIT1 * BOHR * rcrit
    rv2 = rvdw * rvdw
    rv6 = rv2 * rv2 * rv2
    rv10 = rv6 * rv2 * rv2
    rc2 = CUT_OFF * CUT_OFF
    ro = 0.66 * 0.66 * rc2
    cut = jnp.where(
        r < ro, 1.0,
        (rc2 - r) * (rc2 - r) * (rc2 + 2.0 * r - 3.0 * ro) * (1.0 / (rc2 - ro) ** 3))
    r3 = r * r * r
    r4 = r3 * r
    r5 = r4 * r
    b2 = BOHR * BOHR
    b6 = b2 * b2 * b2
    b8 = b6 * b2
    b10 = b8 * b2
    e = -(c6 / (r3 + rv6) * b6 + c8 / (r4 + rv6) * b8
          + c10 / (r5 + rv10) * b10) * cut
    return e


def _edge_body(f0, f1, f2, f3, f4, f5, f6, sidx, didx, out,
               sh, sidx_v, didx_v, sfb, dfb, acc, sem0, sem1):
    feat = [f0, f1, f2, f3, f4, f5, f6]
    sems = [sem0, sem1]
    sid = lax.axis_index("s")
    wid = sid * 2 + lax.axis_index("c")

    @pl.when(sid == 0)
    def _():
        for f in range(7):
            pltpu.sync_copy(feat[f], sh[f])
    plsc.subcore_barrier()

    base_row = wid * _CHUNKS
    pltpu.sync_copy(sidx.at[pl.ds(base_row, _CHUNKS)], sidx_v)
    pltpu.sync_copy(didx.at[pl.ds(base_row, _CHUNKS)], didx_v)
    acc[...] = jnp.zeros((16,), jnp.float32)
    lane = lax.iota(jnp.int32, 16)
    ebase0 = wid * _EPW

    def fire(j, b):
        for f in range(7):
            pltpu.async_copy(sh[f].at[sidx_v.at[j]], sfb[f].at[b], sems[b])
            pltpu.async_copy(sh[f].at[didx_v.at[j]], dfb[f].at[b], sems[b])

    def drain(b):
        for f in range(7):
            pltpu.make_async_copy(
                sh[f].at[pl.ds(0, _CW)], sfb[f].at[b], sems[b]).wait()
            pltpu.make_async_copy(
                sh[f].at[pl.ds(0, _CW)], dfb[f].at[b], sems[b]).wait()

    fire(0, 0)
    fire(1, 1)

    def outer(i, carry):
        for b in range(2):
            j = 2 * i + b
            drain(b)
            a = acc[...]
            for u in range(_CW // 16):
                sl = pl.ds(u * 16, 16)
                fs = [sfb[f][b, sl] for f in range(7)]
                fd = [dfb[f][b, sl] for f in range(7)]
                e = _pair_energy(*fs, *fd)
                eid = ebase0 + j * _CW + u * 16 + lane
                a = a + jnp.where(eid < _E, e, 0.0)
            acc[...] = a

            @pl.when(j + 2 < _CHUNKS)
            def _():
                fire(j + 2, b)
        return carry

    lax.fori_loop(0, _CHUNKS // 2, outer, 0)
    pltpu.sync_copy(acc, out.at[wid])


def _edge_stage(feats, sidxp, didxp):
    mesh = plsc.VectorSubcoreMesh(core_axis_name="c", subcore_axis_name="s")
    fn = functools.partial(
        pl.kernel,
        mesh=mesh,
        out_type=jax.ShapeDtypeStruct((_NW, 16), jnp.float32),
        scratch_types=[
            [pltpu.VMEM_SHARED((_NPAD,), jnp.float32) for _ in range(7)],
            pltpu.VMEM((_CHUNKS, _CW), jnp.int32),
            pltpu.VMEM((_CHUNKS, _CW), jnp.int32),
            [pltpu.VMEM((2, _CW), jnp.float32) for _ in range(7)],
            [pltpu.VMEM((2, _CW), jnp.float32) for _ in range(7)],
            pltpu.VMEM((16,), jnp.float32),
            pltpu.SemaphoreType.DMA,
            pltpu.SemaphoreType.DMA,
        ],
    )(_edge_body)
    return fn(*feats, sidxp, didxp)


def kernel(atomic_index, aev, positions, edge_index, W1, b1, W2, b2,
           v_free, polar_free):
    n, d = aev.shape
    assert n == _N and d == _D
    aev_p = jnp.pad(aev, ((0, _NPAD - _N), (0, 0))).astype(jnp.bfloat16)
    oh = (atomic_index[:, None] == (jnp.arange(_GS, dtype=jnp.int32)[None, :] % 4)
          ).astype(jnp.float32)
    oh_p = jnp.pad(oh, ((0, _NPAD - _N), (0, 0)))
    w1r = jnp.transpose(W1, (2, 0, 1, 3)).reshape(_D, _DH).astype(jnp.bfloat16)
    ratio = polar_free / v_free
    aux = (jnp.zeros((8, _DH), jnp.float32)
           .at[0].set(b1.reshape(_DH))
           .at[1].set(W2.reshape(_DH))
           .at[2, 0:_GS].set(b2.reshape(_GS))
           .at[3, 0:_GS].set(jnp.tile(ratio, 4)))
    m4 = _mlp_stage(aev_p, w1r, aux, oh_p)
    pos_p = jnp.pad(positions, ((0, _NPAD - _N), (0, 0)))
    feats = [m4[:, 0], m4[:, 1], m4[:, 2], m4[:, 3],
             pos_p[:, 0], pos_p[:, 1], pos_p[:, 2]]
    sidxp = jnp.pad(edge_index[0], (0, _EPAD - _E)).reshape(
        _NW * _CHUNKS, _CW)
    didxp = jnp.pad(edge_index[1], (0, _EPAD - _E)).reshape(
        _NW * _CHUNKS, _CW)
    parts = _edge_stage(feats, sidxp, didxp)
    return jnp.sum(parts)


# speedup vs baseline: 2.4420x; 1.0327x over previous
import functools

import jax
import jax.numpy as jnp
import numpy as np
from jax import lax
from jax.experimental import pallas as pl
from jax.experimental.pallas import tpu as pltpu
from jax.experimental.pallas import tpu_sc as plsc

BOHR = 0.529177
CUT_OFF = 20.0
CRIT0 = 0.63
CRIT1 = 1.26

_N = 10000
_NPAD = 10240
_BN = 256
_D = 256
_H = 128
_GS = 16
_DH = 2048
_E = 160000
_NW = 32
_CW = 128
_CHUNKS = 40
_EPW = _CHUNKS * _CW
_EPAD = _NW * _EPW


def _mlp_body(a_ref, w1_ref, e2_ref, aux_ref, oh_ref, out_ref):
    a = a_ref[...]
    w1 = w1_ref[...]
    b1 = aux_ref[0:1, :]
    h = jnp.tanh(jnp.dot(a, w1, preferred_element_type=jnp.float32)
                 .astype(jnp.float32) + b1).astype(jnp.bfloat16)
    out16 = jnp.dot(h, e2_ref[...], preferred_element_type=jnp.float32)
    out16 = out16 + aux_ref[2:3, 0:_GS]
    oh = oh_ref[...]
    sel = out16 * oh
    colg = lax.broadcasted_iota(jnp.int32, (1, _GS), 1) // 4
    mg = []
    for g in range(4):
        mg.append(jnp.sum(jnp.where(colg == g, sel, 0.0), axis=1, keepdims=True))

    def softplus(x):
        return jnp.maximum(x, 0.0) + jnp.log(1.0 + jnp.exp(-jnp.abs(x)))

    m1 = softplus(mg[0]) + 1e-3
    m2 = softplus(mg[1]) + 1e-3
    m3 = softplus(mg[2]) + 1e-3
    v = softplus(mg[3]) + 1e-3
    ratio = aux_ref[3:4, 0:_GS]
    rsel = jnp.sum(jnp.where(colg == 0, oh * ratio, 0.0), axis=1, keepdims=True)
    polar = rsel * v
    out_ref[...] = jnp.concatenate([m1, m2, m3, polar], axis=1)


def _mlp_stage(aev_p, w1r, e2, aux, oh_p):
    return pl.pallas_call(
        _mlp_body,
        grid=(_NPAD // _BN,),
        in_specs=[
            pl.BlockSpec((_BN, _D), lambda i: (i, 0)),
            pl.BlockSpec((_D, _DH), lambda i: (0, 0)),
            pl.BlockSpec((_DH, _GS), lambda i: (0, 0)),
            pl.BlockSpec((8, _DH), lambda i: (0, 0)),
            pl.BlockSpec((_BN, _GS), lambda i: (i, 0)),
        ],
        out_specs=pl.BlockSpec((_BN, 4), lambda i: (i, 0)),
        out_shape=jax.ShapeDtypeStruct((_NPAD, 4), jnp.float32),
    )(aev_p, w1r, e2, aux, oh_p)


def _sqrt16(x):
    b = lax.bitcast_convert_type(x, jnp.int32)
    y = lax.bitcast_convert_type((b >> 1) + jnp.int32(0x1FBD1DF5), jnp.float32)
    y = 0.5 * (y + x / y)
    y = 0.5 * (y + x / y)
    y = 0.5 * (y + x / y)
    return y


def _transpose16(rows, nout=7):
    lane = lax.iota(jnp.int32, 16)
    stages = (8, 4, 2, 1)
    needed = [set(range(nout))]
    for o in reversed(stages):
        needed.append({f ^ b for f in needed[-1] for b in (0, o)})
    needed = needed[::-1]
    cur = {i: rows[i] for i in needed[0]}
    for s, o in enumerate(stages):
        mask = ((lane // o) % 2) == 0
        idx_r = (lane - o) & 15
        idx_l = (lane + o) & 15
        new = {}
        for i in needed[s + 1]:
            j = i ^ o
            if i < j:
                rot = cur[j].at[idx_r].get(mode="promise_in_bounds")
                new[i] = jnp.where(mask, cur[i], rot)
            else:
                rot = cur[j].at[idx_l].get(mode="promise_in_bounds")
                new[i] = jnp.where(mask, rot, cur[i])
        cur = new
    return [cur[f] for f in range(nout)]


def _pair_energy(m1s, m2s, m3s, ps, xs, ys, zs,
                 m1d, m2d, m3d, pd, xd, yd, zd):
    dx = xd - xs
    dy = yd - ys
    dz = zd - zs
    r = dx * dx + dy * dy + dz * dz + 1e-12
    scaled = m1s / ps + m1d / pd
    c6 = m1s * m1d / scaled
    c8 = 1.5 * (m1s * m2d + m2s * m1d) / scaled
    c10 = 2.0 * (m1s * m3d + m3s * m1d + 2.1 * m2s * m2d) / scaled
    rcrit = (_sqrt16(c8 / c6) + _sqrt16(_sqrt16(c10 / c6))
             + _sqrt16(c10 / c8)) * (1.0 / 3.0)
    rvdw = CRIT0 + CRIT1 * BOHR * rcrit
    rv2 = rvdw * rvdw
    rv6 = rv2 * rv2 * rv2
    rv10 = rv6 * rv2 * rv2
    rc2 = CUT_OFF * CUT_OFF
    ro = 0.66 * 0.66 * rc2
    cut = jnp.where(
        r < ro, 1.0,
        (rc2 - r) * (rc2 - r) * (rc2 + 2.0 * r - 3.0 * ro) * (1.0 / (rc2 - ro) ** 3))
    r3 = r * r * r
    r4 = r3 * r
    r5 = r4 * r
    b2 = BOHR * BOHR
    b6 = b2 * b2 * b2
    b8 = b6 * b2
    b10 = b8 * b2
    e = -(c6 / (r3 + rv6) * b6 + c8 / (r4 + rv6) * b8
          + c10 / (r5 + rv10) * b10) * cut
    return e


def _edge_body(f0, f1, f2, f3, f4, f5, f6, sidx, didx, out,
               sh, sidx_v, didx_v, sfb, dfb, acc, sem0, sem1):
    feat = [f0, f1, f2, f3, f4, f5, f6]
    sems = [sem0, sem1]
    sid = lax.axis_index("s")
    wid = sid * 2 + lax.axis_index("c")

    @pl.when(sid == 0)
    def _():
        for f in range(7):
            pltpu.sync_copy(feat[f], sh[f])
    plsc.subcore_barrier()

    base_row = wid * _CHUNKS
    pltpu.sync_copy(sidx.at[pl.ds(base_row, _CHUNKS)], sidx_v)
    pltpu.sync_copy(didx.at[pl.ds(base_row, _CHUNKS)], didx_v)
    acc[...] = jnp.zeros((16,), jnp.float32)
    lane = lax.iota(jnp.int32, 16)
    ebase0 = wid * _EPW

    def fire(j, b):
        for f in range(7):
            pltpu.async_copy(sh[f].at[sidx_v.at[j]], sfb[f].at[b], sems[b])
            pltpu.async_copy(sh[f].at[didx_v.at[j]], dfb[f].at[b], sems[b])

    def drain(b):
        for f in range(7):
            pltpu.make_async_copy(
                sh[f].at[pl.ds(0, _CW)], sfb[f].at[b], sems[b]).wait()
            pltpu.make_async_copy(
                sh[f].at[pl.ds(0, _CW)], dfb[f].at[b], sems[b]).wait()

    fire(0, 0)
    fire(1, 1)

    def outer(i, carry):
        for b in range(2):
            j = 2 * i + b
            drain(b)
            a = acc[...]
            for u in range(_CW // 16):
                sl = pl.ds(u * 16, 16)
                fs = [sfb[f][b, sl] for f in range(7)]
                fd = [dfb[f][b, sl] for f in range(7)]
                e = _pair_energy(*fs, *fd)
                eid = ebase0 + j * _CW + u * 16 + lane
                a = a + jnp.where(eid < _E, e, 0.0)
            acc[...] = a

            @pl.when(j + 2 < _CHUNKS)
            def _():
                fire(j + 2, b)
        return carry

    lax.fori_loop(0, _CHUNKS // 2, outer, 0)
    pltpu.sync_copy(acc, out.at[wid])


def _edge_stage(feats, sidxp, didxp):
    mesh = plsc.VectorSubcoreMesh(core_axis_name="c", subcore_axis_name="s")
    fn = functools.partial(
        pl.kernel,
        mesh=mesh,
        out_type=jax.ShapeDtypeStruct((_NW, 16), jnp.float32),
        scratch_types=[
            [pltpu.VMEM_SHARED((_NPAD,), jnp.float32) for _ in range(7)],
            pltpu.VMEM((_CHUNKS, _CW), jnp.int32),
            pltpu.VMEM((_CHUNKS, _CW), jnp.int32),
            [pltpu.VMEM((2, _CW), jnp.float32) for _ in range(7)],
            [pltpu.VMEM((2, _CW), jnp.float32) for _ in range(7)],
            pltpu.VMEM((16,), jnp.float32),
            pltpu.SemaphoreType.DMA,
            pltpu.SemaphoreType.DMA,
        ],
    )(_edge_body)
    return fn(*feats, sidxp, didxp)


def kernel(atomic_index, aev, positions, edge_index, W1, b1, W2, b2,
           v_free, polar_free):
    n, d = aev.shape
    assert n == _N and d == _D
    aev_p = jnp.pad(aev, ((0, _NPAD - _N), (0, 0))).astype(jnp.bfloat16)
    oh = (atomic_index[:, None] == (jnp.arange(_GS, dtype=jnp.int32)[None, :] % 4)
          ).astype(jnp.float32)
    oh_p = jnp.pad(oh, ((0, _NPAD - _N), (0, 0)))
    w1r = jnp.transpose(W1, (2, 0, 1, 3)).reshape(_D, _DH).astype(jnp.bfloat16)
    ratio = polar_free / v_free
    aux = (jnp.zeros((8, _DH), jnp.float32)
           .at[0].set(b1.reshape(_DH))
           .at[2, 0:_GS].set(b2.reshape(_GS))
           .at[3, 0:_GS].set(jnp.tile(ratio, 4)))
    e2 = jnp.where(
        jnp.arange(_DH)[:, None] // _H == jnp.arange(_GS)[None, :],
        W2.reshape(_DH, 1), 0.0).astype(jnp.bfloat16)
    m4 = _mlp_stage(aev_p, w1r, e2, aux, oh_p)
    pos_p = jnp.pad(positions, ((0, _NPAD - _N), (0, 0)))
    feats = [m4[:, 0], m4[:, 1], m4[:, 2], m4[:, 3],
             pos_p[:, 0], pos_p[:, 1], pos_p[:, 2]]
    sidxp = jnp.pad(edge_index[0], (0, _EPAD - _E)).reshape(
        _NW * _CHUNKS, _CW)
    didxp = jnp.pad(edge_index[1], (0, _EPAD - _E)).reshape(
        _NW * _CHUNKS, _CW)
    parts = _edge_stage(feats, sidxp, didxp)
    return jnp.sum(parts)


# speedup vs baseline: 2.4456x; 1.0015x over previous
import functools

import jax
import jax.numpy as jnp
import numpy as np
from jax import lax
from jax.experimental import pallas as pl
from jax.experimental.pallas import tpu as pltpu
from jax.experimental.pallas import tpu_sc as plsc

BOHR = 0.529177
CUT_OFF = 20.0
CRIT0 = 0.63
CRIT1 = 1.26

_N = 10000
_NPAD = 10240
_BN = 256
_D = 256
_H = 128
_GS = 16
_DH = 2048
_E = 160000
_NW = 32
_CW = 128
_CHUNKS = 40
_CROWS = 40
_EPW = _CHUNKS * _CW
_EPAD = _NW * _EPW


def _mlp_body(a_ref, w1_ref, e2_ref, aux_ref, oh_ref, out_ref):
    a = a_ref[...]
    w1 = w1_ref[...]
    b1 = aux_ref[0:1, :]
    h = jnp.tanh(jnp.dot(a, w1, preferred_element_type=jnp.float32)
                 .astype(jnp.float32) + b1).astype(jnp.bfloat16)
    out16 = jnp.dot(h, e2_ref[...], preferred_element_type=jnp.float32)
    out16 = out16 + aux_ref[2:3, 0:_GS]
    oh = oh_ref[...]
    sel = out16 * oh
    colg = lax.broadcasted_iota(jnp.int32, (1, _GS), 1) // 4
    mg = []
    for g in range(4):
        mg.append(jnp.sum(jnp.where(colg == g, sel, 0.0), axis=1, keepdims=True))

    def softplus(x):
        return jnp.maximum(x, 0.0) + jnp.log(1.0 + jnp.exp(-jnp.abs(x)))

    m1 = softplus(mg[0]) + 1e-3
    m2 = softplus(mg[1]) + 1e-3
    m3 = softplus(mg[2]) + 1e-3
    v = softplus(mg[3]) + 1e-3
    ratio = aux_ref[3:4, 0:_GS]
    rsel = jnp.sum(jnp.where(colg == 0, oh * ratio, 0.0), axis=1, keepdims=True)
    polar = rsel * v
    out_ref[...] = jnp.concatenate([m1, m2, m3, polar], axis=1)


def _mlp_stage(aev_p, w1r, e2, aux, oh_p):
    return pl.pallas_call(
        _mlp_body,
        grid=(_NPAD // _BN,),
        in_specs=[
            pl.BlockSpec((_BN, _D), lambda i: (i, 0)),
            pl.BlockSpec((_D, _DH), lambda i: (0, 0)),
            pl.BlockSpec((_DH, _GS), lambda i: (0, 0)),
            pl.BlockSpec((8, _DH), lambda i: (0, 0)),
            pl.BlockSpec((_BN, _GS), lambda i: (i, 0)),
        ],
        out_specs=pl.BlockSpec((_BN, 4), lambda i: (i, 0)),
        out_shape=jax.ShapeDtypeStruct((_NPAD, 4), jnp.float32),
    )(aev_p, w1r, e2, aux, oh_p)


def _sqrt16(x):
    b = lax.bitcast_convert_type(x, jnp.int32)
    y = lax.bitcast_convert_type((b >> 1) + jnp.int32(0x1FBD1DF5), jnp.float32)
    y = 0.5 * (y + x / y)
    y = 0.5 * (y + x / y)
    y = 0.5 * (y + x / y)
    return y


def _transpose16(rows, nout=7):
    lane = lax.iota(jnp.int32, 16)
    stages = (8, 4, 2, 1)
    needed = [set(range(nout))]
    for o in reversed(stages):
        needed.append({f ^ b for f in needed[-1] for b in (0, o)})
    needed = needed[::-1]
    cur = {i: rows[i] for i in needed[0]}
    for s, o in enumerate(stages):
        mask = ((lane // o) % 2) == 0
        idx_r = (lane - o) & 15
        idx_l = (lane + o) & 15
        new = {}
        for i in needed[s + 1]:
            j = i ^ o
            if i < j:
                rot = cur[j].at[idx_r].get(mode="promise_in_bounds")
                new[i] = jnp.where(mask, cur[i], rot)
            else:
                rot = cur[j].at[idx_l].get(mode="promise_in_bounds")
                new[i] = jnp.where(mask, rot, cur[i])
        cur = new
    return [cur[f] for f in range(nout)]


def _pair_energy(m1s, m2s, m3s, ps, xs, ys, zs,
                 m1d, m2d, m3d, pd, xd, yd, zd):
    dx = xd - xs
    dy = yd - ys
    dz = zd - zs
    r = dx * dx + dy * dy + dz * dz + 1e-12
    scaled = m1s / ps + m1d / pd
    c6 = m1s * m1d / scaled
    c8 = 1.5 * (m1s * m2d + m2s * m1d) / scaled
    c10 = 2.0 * (m1s * m3d + m3s * m1d + 2.1 * m2s * m2d) / scaled
    rcrit = (_sqrt16(c8 / c6) + _sqrt16(_sqrt16(c10 / c6))
             + _sqrt16(c10 / c8)) * (1.0 / 3.0)
    rvdw = CRIT0 + CRIT1 * BOHR * rcrit
    rv2 = rvdw * rvdw
    rv6 = rv2 * rv2 * rv2
    rv10 = rv6 * rv2 * rv2
    rc2 = CUT_OFF * CUT_OFF
    ro = 0.66 * 0.66 * rc2
    cut = jnp.where(
        r < ro, 1.0,
        (rc2 - r) * (rc2 - r) * (rc2 + 2.0 * r - 3.0 * ro) * (1.0 / (rc2 - ro) ** 3))
    r3 = r * r * r
    r4 = r3 * r
    r5 = r4 * r
    b2 = BOHR * BOHR
    b6 = b2 * b2 * b2
    b8 = b6 * b2
    b10 = b8 * b2
    e = -(c6 / (r3 + rv6) * b6 + c8 / (r4 + rv6) * b8
          + c10 / (r5 + rv10) * b10) * cut
    return e


def _edge_body(f0, f1, f2, f3, f4, f5, f6, sidx, didx, out,
               sh, sidx_v, didx_v, sfb, dfb, acc, sem0, sem1):
    feat = [f0, f1, f2, f3, f4, f5, f6]
    sems = [sem0, sem1]
    sid = lax.axis_index("s")
    wid = sid * 2 + lax.axis_index("c")

    @pl.when(sid == 0)
    def _():
        for f in range(7):
            pltpu.sync_copy(feat[f], sh[f])
    plsc.subcore_barrier()

    pltpu.sync_copy(sidx.at[pl.ds(wid * _CROWS, _CROWS)], sidx_v)
    pltpu.sync_copy(didx.at[pl.ds(wid * _CROWS, _CROWS)], didx_v)
    acc[...] = jnp.zeros((16,), jnp.float32)
    lane = lax.iota(jnp.int32, 16)
    ebase0 = wid * _EPW

    def fire(j, b):
        for f in range(7):
            pltpu.async_copy(sh[f].at[sidx_v.at[j]], sfb[f].at[b], sems[b])
            pltpu.async_copy(sh[f].at[didx_v.at[j]], dfb[f].at[b], sems[b])

    def drain(b):
        for f in range(7):
            pltpu.make_async_copy(
                sh[f].at[pl.ds(0, _CW)], sfb[f].at[b], sems[b]).wait()
            pltpu.make_async_copy(
                sh[f].at[pl.ds(0, _CW)], dfb[f].at[b], sems[b]).wait()

    fire(0, 0)
    fire(1, 1)

    def outer(i, carry):
        for b in range(2):
            j = 2 * i + b
            drain(b)
            a = acc[...]
            for u in range(_CW // 16):
                sl = pl.ds(u * 16, 16)
                fs = [sfb[f][b, sl] for f in range(7)]
                fd = [dfb[f][b, sl] for f in range(7)]
                e = _pair_energy(*fs, *fd)
                eid = ebase0 + j * _CW + u * 16 + lane
                a = a + jnp.where(eid < _E, e, 0.0)
            acc[...] = a

            @pl.when(j + 2 < _CHUNKS)
            def _():
                fire(j + 2, b)
        return carry

    lax.fori_loop(0, _CHUNKS // 2, outer, 0)
    pltpu.sync_copy(acc, out.at[wid])


def _edge_stage(feats, sidxp, didxp):
    mesh = plsc.VectorSubcoreMesh(core_axis_name="c", subcore_axis_name="s")
    fn = functools.partial(
        pl.kernel,
        mesh=mesh,
        out_type=jax.ShapeDtypeStruct((_NW, 16), jnp.float32),
        scratch_types=[
            [pltpu.VMEM_SHARED((_NPAD,), jnp.float32) for _ in range(7)],
            pltpu.VMEM((_CROWS, _CW), jnp.int32),
            pltpu.VMEM((_CROWS, _CW), jnp.int32),
            [pltpu.VMEM((2, _CW), jnp.float32) for _ in range(7)],
            [pltpu.VMEM((2, _CW), jnp.float32) for _ in range(7)],
            pltpu.VMEM((16,), jnp.float32),
            pltpu.SemaphoreType.DMA,
            pltpu.SemaphoreType.DMA,
        ],
    )(_edge_body)
    return fn(*feats, sidxp, didxp)


def kernel(atomic_index, aev, positions, edge_index, W1, b1, W2, b2,
           v_free, polar_free):
    n, d = aev.shape
    assert n == _N and d == _D
    aev_p = jnp.pad(aev, ((0, _NPAD - _N), (0, 0))).astype(jnp.bfloat16)
    oh = (atomic_index[:, None] == (jnp.arange(_GS, dtype=jnp.int32)[None, :] % 4)
          ).astype(jnp.float32)
    oh_p = jnp.pad(oh, ((0, _NPAD - _N), (0, 0)))
    w1r = jnp.transpose(W1, (2, 0, 1, 3)).reshape(_D, _DH).astype(jnp.bfloat16)
    ratio = polar_free / v_free
    aux = (jnp.zeros((8, _DH), jnp.float32)
           .at[0].set(b1.reshape(_DH))
           .at[2, 0:_GS].set(b2.reshape(_GS))
           .at[3, 0:_GS].set(jnp.tile(ratio, 4)))
    e2 = jnp.where(
        jnp.arange(_DH)[:, None] // _H == jnp.arange(_GS)[None, :],
        W2.reshape(_DH, 1), 0.0).astype(jnp.bfloat16)
    m4 = _mlp_stage(aev_p, w1r, e2, aux, oh_p)
    pos_p = jnp.pad(positions, ((0, _NPAD - _N), (0, 0)))
    feats = [m4[:, 0], m4[:, 1], m4[:, 2], m4[:, 3],
             pos_p[:, 0], pos_p[:, 1], pos_p[:, 2]]
    def _idx_rows(x):
        r = jnp.pad(x, (0, _EPAD - _E)).reshape(_NW, _CHUNKS, _CW)
        return jnp.pad(r, ((0, 0), (0, _CROWS - _CHUNKS), (0, 0))
                       ).reshape(_NW * _CROWS, _CW)
    sidxp = _idx_rows(edge_index[0])
    didxp = _idx_rows(edge_index[1])
    parts = _edge_stage(feats, sidxp, didxp)
    return jnp.sum(parts)


# speedup vs baseline: 2.4715x; 1.0106x over previous
import functools

import jax
import jax.numpy as jnp
import numpy as np
from jax import lax
from jax.experimental import pallas as pl
from jax.experimental.pallas import tpu as pltpu
from jax.experimental.pallas import tpu_sc as plsc

BOHR = 0.529177
CUT_OFF = 20.0
CRIT0 = 0.63
CRIT1 = 1.26

_N = 10000
_NPAD = 10240
_BN = 256
_D = 256
_H = 128
_GS = 16
_DH = 2048
_E = 160000
_NW = 32
_CW = 128
_CHUNKS = 40
_CROWS = 40
_EPW = _CHUNKS * _CW
_EPAD = _NW * _EPW


def _mlp_body(a_ref, w1_ref, e2_ref, aux_ref, oh_ref, out_ref):
    a = a_ref[...]
    w1 = w1_ref[...]
    b1 = aux_ref[0:1, :]
    h = jnp.tanh(jnp.dot(a, w1, preferred_element_type=jnp.float32)
                 .astype(jnp.float32) + b1).astype(jnp.bfloat16)
    out16 = jnp.dot(h, e2_ref[...], preferred_element_type=jnp.float32)
    out16 = out16 + aux_ref[2:3, 0:_GS]
    oh = oh_ref[...]
    sel = out16 * oh
    colg = lax.broadcasted_iota(jnp.int32, (1, _GS), 1) // 4
    mg = []
    for g in range(4):
        mg.append(jnp.sum(jnp.where(colg == g, sel, 0.0), axis=1, keepdims=True))

    def softplus(x):
        return jnp.maximum(x, 0.0) + jnp.log(1.0 + jnp.exp(-jnp.abs(x)))

    m1 = softplus(mg[0]) + 1e-3
    m2 = softplus(mg[1]) + 1e-3
    m3 = softplus(mg[2]) + 1e-3
    v = softplus(mg[3]) + 1e-3
    ratio = aux_ref[3:4, 0:_GS]
    rsel = jnp.sum(jnp.where(colg == 0, oh * ratio, 0.0), axis=1, keepdims=True)
    polar = rsel * v
    out_ref[...] = jnp.concatenate([m1, m2, m3, polar], axis=1)


def _mlp_stage(aev_p, w1r, e2, aux, oh_p):
    return pl.pallas_call(
        _mlp_body,
        grid=(_NPAD // _BN,),
        in_specs=[
            pl.BlockSpec((_BN, _D), lambda i: (i, 0)),
            pl.BlockSpec((_D, _DH), lambda i: (0, 0)),
            pl.BlockSpec((_DH, _GS), lambda i: (0, 0)),
            pl.BlockSpec((8, _DH), lambda i: (0, 0)),
            pl.BlockSpec((_BN, _GS), lambda i: (i, 0)),
        ],
        out_specs=pl.BlockSpec((_BN, 4), lambda i: (i, 0)),
        out_shape=jax.ShapeDtypeStruct((_NPAD, 4), jnp.float32),
    )(aev_p, w1r, e2, aux, oh_p)


def _sqrt16(x):
    b = lax.bitcast_convert_type(x, jnp.int32)
    y = lax.bitcast_convert_type((b >> 1) + jnp.int32(0x1FBD1DF5), jnp.float32)
    y = 0.5 * (y + x / y)
    y = 0.5 * (y + x / y)
    y = 0.5 * (y + x / y)
    return y


def _transpose16(rows, nout=7):
    lane = lax.iota(jnp.int32, 16)
    stages = (8, 4, 2, 1)
    needed = [set(range(nout))]
    for o in reversed(stages):
        needed.append({f ^ b for f in needed[-1] for b in (0, o)})
    needed = needed[::-1]
    cur = {i: rows[i] for i in needed[0]}
    for s, o in enumerate(stages):
        mask = ((lane // o) % 2) == 0
        idx_r = (lane - o) & 15
        idx_l = (lane + o) & 15
        new = {}
        for i in needed[s + 1]:
            j = i ^ o
            if i < j:
                rot = cur[j].at[idx_r].get(mode="promise_in_bounds")
                new[i] = jnp.where(mask, cur[i], rot)
            else:
                rot = cur[j].at[idx_l].get(mode="promise_in_bounds")
                new[i] = jnp.where(mask, rot, cur[i])
        cur = new
    return [cur[f] for f in range(nout)]


def _pair_energy(m1s, m2s, m3s, ps, xs, ys, zs,
                 m1d, m2d, m3d, pd, xd, yd, zd):
    dx = xd - xs
    dy = yd - ys
    dz = zd - zs
    r = dx * dx + dy * dy + dz * dz + 1e-12
    scaled = m1s / ps + m1d / pd
    c6 = m1s * m1d / scaled
    c8 = 1.5 * (m1s * m2d + m2s * m1d) / scaled
    c10 = 2.0 * (m1s * m3d + m3s * m1d + 2.1 * m2s * m2d) / scaled
    rcrit = (_sqrt16(c8 / c6) + _sqrt16(_sqrt16(c10 / c6))
             + _sqrt16(c10 / c8)) * (1.0 / 3.0)
    rvdw = CRIT0 + CRIT1 * BOHR * rcrit
    rv2 = rvdw * rvdw
    rv6 = rv2 * rv2 * rv2
    rv10 = rv6 * rv2 * rv2
    rc2 = CUT_OFF * CUT_OFF
    ro = 0.66 * 0.66 * rc2
    cut = jnp.where(
        r < ro, 1.0,
        (rc2 - r) * (rc2 - r) * (rc2 + 2.0 * r - 3.0 * ro) * (1.0 / (rc2 - ro) ** 3))
    r3 = r * r * r
    r4 = r3 * r
    r5 = r4 * r
    b2 = BOHR * BOHR
    b6 = b2 * b2 * b2
    b8 = b6 * b2
    b10 = b8 * b2
    e = -(c6 / (r3 + rv6) * b6 + c8 / (r4 + rv6) * b8
          + c10 / (r5 + rv10) * b10) * cut
    return e


def _unpack2(w):
    lo = lax.bitcast_convert_type(w << 16, jnp.float32)
    hi = lax.bitcast_convert_type(w & jnp.int32(-65536), jnp.float32)
    return lo, hi


def _edge_body(f0, f1, f2, f3, sidx, didx, out,
               sh, sidx_v, didx_v, sfb, dfb, acc, sem0, sem1):
    feat = [f0, f1, f2, f3]
    sems = [sem0, sem1]
    sid = lax.axis_index("s")
    wid = sid * 2 + lax.axis_index("c")

    @pl.when(sid == 0)
    def _():
        for f in range(4):
            pltpu.sync_copy(feat[f], sh[f])
    plsc.subcore_barrier()

    pltpu.sync_copy(sidx.at[pl.ds(wid * _CROWS, _CROWS)], sidx_v)
    pltpu.sync_copy(didx.at[pl.ds(wid * _CROWS, _CROWS)], didx_v)
    acc[...] = jnp.zeros((16,), jnp.float32)
    lane = lax.iota(jnp.int32, 16)
    ebase0 = wid * _EPW

    def fire(j, b):
        for f in range(4):
            pltpu.async_copy(sh[f].at[sidx_v.at[j]], sfb[f].at[b], sems[b])
            pltpu.async_copy(sh[f].at[didx_v.at[j]], dfb[f].at[b], sems[b])

    def drain(b):
        for f in range(4):
            pltpu.make_async_copy(
                sh[f].at[pl.ds(0, _CW)], sfb[f].at[b], sems[b]).wait()
            pltpu.make_async_copy(
                sh[f].at[pl.ds(0, _CW)], dfb[f].at[b], sems[b]).wait()

    fire(0, 0)
    fire(1, 1)

    def outer(i, carry):
        for b in range(2):
            j = 2 * i + b
            drain(b)
            a = acc[...]
            for u in range(_CW // 16):
                sl = pl.ds(u * 16, 16)

                def unp(fb):
                    m1, m2 = _unpack2(fb[0][b, sl])
                    m3, po = _unpack2(fb[1][b, sl])
                    x, y = _unpack2(fb[2][b, sl])
                    z = fb[3][b, sl]
                    return [m1, m2, m3, po, x, y, z]

                e = _pair_energy(*unp(sfb), *unp(dfb))
                eid = ebase0 + j * _CW + u * 16 + lane
                a = a + jnp.where(eid < _E, e, 0.0)
            acc[...] = a

            @pl.when(j + 2 < _CHUNKS)
            def _():
                fire(j + 2, b)
        return carry

    lax.fori_loop(0, _CHUNKS // 2, outer, 0)
    pltpu.sync_copy(acc, out.at[wid])


def _edge_stage(feats, sidxp, didxp):
    mesh = plsc.VectorSubcoreMesh(core_axis_name="c", subcore_axis_name="s")
    fn = functools.partial(
        pl.kernel,
        mesh=mesh,
        out_type=jax.ShapeDtypeStruct((_NW, 16), jnp.float32),
        scratch_types=[
            [pltpu.VMEM_SHARED((_NPAD,), jnp.int32) for _ in range(3)]
            + [pltpu.VMEM_SHARED((_NPAD,), jnp.float32)],
            pltpu.VMEM((_CROWS, _CW), jnp.int32),
            pltpu.VMEM((_CROWS, _CW), jnp.int32),
            [pltpu.VMEM((2, _CW), jnp.int32) for _ in range(3)]
            + [pltpu.VMEM((2, _CW), jnp.float32)],
            [pltpu.VMEM((2, _CW), jnp.int32) for _ in range(3)]
            + [pltpu.VMEM((2, _CW), jnp.float32)],
            pltpu.VMEM((16,), jnp.float32),
            pltpu.SemaphoreType.DMA,
            pltpu.SemaphoreType.DMA,
        ],
    )(_edge_body)
    return fn(*feats, sidxp, didxp)


def kernel(atomic_index, aev, positions, edge_index, W1, b1, W2, b2,
           v_free, polar_free):
    n, d = aev.shape
    assert n == _N and d == _D
    aev_p = jnp.pad(aev, ((0, _NPAD - _N), (0, 0))).astype(jnp.bfloat16)
    oh = (atomic_index[:, None] == (jnp.arange(_GS, dtype=jnp.int32)[None, :] % 4)
          ).astype(jnp.float32)
    oh_p = jnp.pad(oh, ((0, _NPAD - _N), (0, 0)))
    w1r = jnp.transpose(W1, (2, 0, 1, 3)).reshape(_D, _DH).astype(jnp.bfloat16)
    ratio = polar_free / v_free
    aux = (jnp.zeros((8, _DH), jnp.float32)
           .at[0].set(b1.reshape(_DH))
           .at[2, 0:_GS].set(b2.reshape(_GS))
           .at[3, 0:_GS].set(jnp.tile(ratio, 4)))
    e2 = jnp.where(
        jnp.arange(_DH)[:, None] // _H == jnp.arange(_GS)[None, :],
        W2.reshape(_DH, 1), 0.0).astype(jnp.bfloat16)
    m4 = _mlp_stage(aev_p, w1r, e2, aux, oh_p)
    pos_p = jnp.pad(positions, ((0, _NPAD - _N), (0, 0)))

    def _pack2(lo, hi):
        lo16 = lax.bitcast_convert_type(
            lo.astype(jnp.bfloat16), jnp.uint16).astype(jnp.uint32)
        hi16 = lax.bitcast_convert_type(
            hi.astype(jnp.bfloat16), jnp.uint16).astype(jnp.uint32)
        return lax.bitcast_convert_type(lo16 | (hi16 << 16), jnp.int32)

    feats = [_pack2(m4[:, 0], m4[:, 1]),
             _pack2(m4[:, 2], m4[:, 3]),
             _pack2(pos_p[:, 0], pos_p[:, 1]),
             pos_p[:, 2]]
    def _idx_rows(x):
        r = jnp.pad(x, (0, _EPAD - _E)).reshape(_NW, _CHUNKS, _CW)
        return jnp.pad(r, ((0, 0), (0, _CROWS - _CHUNKS), (0, 0))
                       ).reshape(_NW * _CROWS, _CW)
    sidxp = _idx_rows(edge_index[0])
    didxp = _idx_rows(edge_index[1])
    parts = _edge_stage(feats, sidxp, didxp)
    return jnp.sum(parts)


# speedup vs baseline: 2.4883x; 1.0068x over previous
import functools

import jax
import jax.numpy as jnp
from jax import lax
from jax.experimental import pallas as pl
from jax.experimental.pallas import tpu as pltpu
from jax.experimental.pallas import tpu_sc as plsc

BOHR = 0.529177
CUT_OFF = 20.0
CRIT0 = 0.63
CRIT1 = 1.26

_N = 10000
_NPAD = 10240
_BN = 256
_D = 256
_H = 128
_GS = 16
_DH = 2048
_E = 160000
_NW = 32
_CW = 128
_CHUNKS = 40
_CROWS = 40
_EPW = _CHUNKS * _CW
_EPAD = _NW * _EPW


def _mlp_body(a_ref, w1_ref, e2_ref, aux_ref, oh_ref, out_ref):
    a = a_ref[...]
    w1 = w1_ref[...]
    b1 = aux_ref[0:1, :]
    h = jnp.tanh(jnp.dot(a, w1, preferred_element_type=jnp.float32)
                 .astype(jnp.float32) + b1).astype(jnp.bfloat16)
    out16 = jnp.dot(h, e2_ref[...], preferred_element_type=jnp.float32)
    out16 = out16 + aux_ref[2:3, 0:_GS]
    oh = oh_ref[...]
    sel = out16 * oh
    colg = lax.broadcasted_iota(jnp.int32, (1, _GS), 1) // 4
    mg = []
    for g in range(4):
        mg.append(jnp.sum(jnp.where(colg == g, sel, 0.0), axis=1, keepdims=True))

    def softplus(x):
        return jnp.maximum(x, 0.0) + jnp.log(1.0 + jnp.exp(-jnp.abs(x)))

    m1 = softplus(mg[0]) + 1e-3
    m2 = softplus(mg[1]) + 1e-3
    m3 = softplus(mg[2]) + 1e-3
    v = softplus(mg[3]) + 1e-3
    ratio = aux_ref[3:4, 0:_GS]
    rsel = jnp.sum(jnp.where(colg == 0, oh * ratio, 0.0), axis=1, keepdims=True)
    polar = rsel * v
    out_ref[...] = jnp.concatenate([m1, m2, m3, polar], axis=1)


def _mlp_stage(aev_p, w1r, e2, aux, oh_p):
    return pl.pallas_call(
        _mlp_body,
        grid=(_NPAD // _BN,),
        in_specs=[
            pl.BlockSpec((_BN, _D), lambda i: (i, 0)),
            pl.BlockSpec((_D, _DH), lambda i: (0, 0)),
            pl.BlockSpec((_DH, _GS), lambda i: (0, 0)),
            pl.BlockSpec((8, _DH), lambda i: (0, 0)),
            pl.BlockSpec((_BN, _GS), lambda i: (i, 0)),
        ],
        out_specs=pl.BlockSpec((_BN, 4), lambda i: (i, 0)),
        out_shape=jax.ShapeDtypeStruct((_NPAD, 4), jnp.float32),
    )(aev_p, w1r, e2, aux, oh_p)


def _sqrt16(x):
    b = lax.bitcast_convert_type(x, jnp.int32)
    y = lax.bitcast_convert_type((b >> 1) + jnp.int32(0x1FBD1DF5), jnp.float32)
    y = 0.5 * (y + x / y)
    y = 0.5 * (y + x / y)
    y = 0.5 * (y + x / y)
    return y


def _pair_energy(m1s, m2s, m3s, ps, xs, ys, zs,
                 m1d, m2d, m3d, pd, xd, yd, zd):
    dx = xd - xs
    dy = yd - ys
    dz = zd - zs
    r = dx * dx + dy * dy + dz * dz + 1e-12
    scaled = m1s / ps + m1d / pd
    c6 = m1s * m1d / scaled
    c8 = 1.5 * (m1s * m2d + m2s * m1d) / scaled
    c10 = 2.0 * (m1s * m3d + m3s * m1d + 2.1 * m2s * m2d) / scaled
    rcrit = (_sqrt16(c8 / c6) + _sqrt16(_sqrt16(c10 / c6))
             + _sqrt16(c10 / c8)) * (1.0 / 3.0)
    rvdw = CRIT0 + CRIT1 * BOHR * rcrit
    rv2 = rvdw * rvdw
    rv6 = rv2 * rv2 * rv2
    rv10 = rv6 * rv2 * rv2
    rc2 = CUT_OFF * CUT_OFF
    ro = 0.66 * 0.66 * rc2
    cut = jnp.where(
        r < ro, 1.0,
        (rc2 - r) * (rc2 - r) * (rc2 + 2.0 * r - 3.0 * ro) * (1.0 / (rc2 - ro) ** 3))
    r3 = r * r * r
    r4 = r3 * r
    r5 = r4 * r
    b2 = BOHR * BOHR
    b6 = b2 * b2 * b2
    b8 = b6 * b2
    b10 = b8 * b2
    e = -(c6 / (r3 + rv6) * b6 + c8 / (r4 + rv6) * b8
          + c10 / (r5 + rv10) * b10) * cut
    return e


def _unpack2(w):
    lo = lax.bitcast_convert_type(w << 16, jnp.float32)
    hi = lax.bitcast_convert_type(w & jnp.int32(-65536), jnp.float32)
    return lo, hi


def _edge_body(f0, f1, f2, f3, sidx, didx, out,
               sh, sidx_v, didx_v, sfb, dfb, acc, sem0, sem1):
    feat = [f0, f1, f2, f3]
    sems = [sem0, sem1]
    sid = lax.axis_index("s")
    wid = sid * 2 + lax.axis_index("c")

    @pl.when(sid == 0)
    def _():
        for f in range(4):
            pltpu.sync_copy(feat[f], sh[f])
    plsc.subcore_barrier()

    pltpu.sync_copy(sidx.at[pl.ds(wid * _CROWS, _CROWS)], sidx_v)
    pltpu.sync_copy(didx.at[pl.ds(wid * _CROWS, _CROWS)], didx_v)
    acc[...] = jnp.zeros((16,), jnp.float32)
    lane = lax.iota(jnp.int32, 16)
    ebase0 = wid * _EPW

    def fire(j, b):
        for f in range(4):
            pltpu.async_copy(sh[f].at[sidx_v.at[j]], sfb[f].at[b], sems[b])
            pltpu.async_copy(sh[f].at[didx_v.at[j]], dfb[f].at[b], sems[b])

    def drain(b):
        for f in range(4):
            pltpu.make_async_copy(
                sh[f].at[pl.ds(0, _CW)], sfb[f].at[b], sems[b]).wait()
            pltpu.make_async_copy(
                sh[f].at[pl.ds(0, _CW)], dfb[f].at[b], sems[b]).wait()

    fire(0, 0)
    fire(1, 1)

    def outer(i, carry):
        for b in range(2):
            j = 2 * i + b
            drain(b)
            a = acc[...]
            for u in range(_CW // 16):
                sl = pl.ds(u * 16, 16)

                def unp(fb):
                    m1, m2 = _unpack2(fb[0][b, sl])
                    m3, po = _unpack2(fb[1][b, sl])
                    x, y = _unpack2(fb[2][b, sl])
                    z = fb[3][b, sl]
                    return [m1, m2, m3, po, x, y, z]

                e = _pair_energy(*unp(sfb), *unp(dfb))
                eid = ebase0 + j * _CW + u * 16 + lane
                a = a + jnp.where(eid < _E, e, 0.0)
            acc[...] = a

            @pl.when(j + 2 < _CHUNKS)
            def _():
                fire(j + 2, b)
        return carry

    lax.fori_loop(0, _CHUNKS // 2, outer, 0)
    pltpu.sync_copy(acc, out.at[wid])


def _edge_stage(feats, sidxp, didxp):
    mesh = plsc.VectorSubcoreMesh(core_axis_name="c", subcore_axis_name="s")
    fn = functools.partial(
        pl.kernel,
        mesh=mesh,
        out_type=jax.ShapeDtypeStruct((_NW, 16), jnp.float32),
        scratch_types=[
            [pltpu.VMEM_SHARED((_NPAD,), jnp.int32) for _ in range(3)]
            + [pltpu.VMEM_SHARED((_NPAD,), jnp.float32)],
            pltpu.VMEM((_CROWS, _CW), jnp.int32),
            pltpu.VMEM((_CROWS, _CW), jnp.int32),
            [pltpu.VMEM((2, _CW), jnp.int32) for _ in range(3)]
            + [pltpu.VMEM((2, _CW), jnp.float32)],
            [pltpu.VMEM((2, _CW), jnp.int32) for _ in range(3)]
            + [pltpu.VMEM((2, _CW), jnp.float32)],
            pltpu.VMEM((16,), jnp.float32),
            pltpu.SemaphoreType.DMA,
            pltpu.SemaphoreType.DMA,
        ],
    )(_edge_body)
    return fn(*feats, sidxp, didxp)


def kernel(atomic_index, aev, positions, edge_index, W1, b1, W2, b2,
           v_free, polar_free):
    n, d = aev.shape
    assert n == _N and d == _D
    aev_p = jnp.pad(aev, ((0, _NPAD - _N), (0, 0))).astype(jnp.bfloat16)
    oh = (atomic_index[:, None] == (jnp.arange(_GS, dtype=jnp.int32)[None, :] % 4)
          ).astype(jnp.float32)
    oh_p = jnp.pad(oh, ((0, _NPAD - _N), (0, 0)))
    w1r = jnp.transpose(W1, (2, 0, 1, 3)).reshape(_D, _DH).astype(jnp.bfloat16)
    ratio = polar_free / v_free
    aux = (jnp.zeros((8, _DH), jnp.float32)
           .at[0].set(b1.reshape(_DH))
           .at[2, 0:_GS].set(b2.reshape(_GS))
           .at[3, 0:_GS].set(jnp.tile(ratio, 4)))
    e2 = jnp.where(
        jnp.arange(_DH)[:, None] // _H == jnp.arange(_GS)[None, :],
        W2.reshape(_DH, 1), 0.0).astype(jnp.bfloat16)
    m4 = _mlp_stage(aev_p, w1r, e2, aux, oh_p)
    pos_p = jnp.pad(positions, ((0, _NPAD - _N), (0, 0)))

    def _pack2(lo, hi):
        lo16 = lax.bitcast_convert_type(
            lo.astype(jnp.bfloat16), jnp.uint16).astype(jnp.uint32)
        hi16 = lax.bitcast_convert_type(
            hi.astype(jnp.bfloat16), jnp.uint16).astype(jnp.uint32)
        return lax.bitcast_convert_type(lo16 | (hi16 << 16), jnp.int32)

    feats = [_pack2(m4[:, 0], m4[:, 1]),
             _pack2(m4[:, 2], m4[:, 3]),
             _pack2(pos_p[:, 0], pos_p[:, 1]),
             pos_p[:, 2]]
    def _idx_rows(x):
        r = jnp.pad(x, (0, _EPAD - _E)).reshape(_NW, _CHUNKS, _CW)
        return jnp.pad(r, ((0, 0), (0, _CROWS - _CHUNKS), (0, 0))
                       ).reshape(_NW * _CROWS, _CW)
    sidxp = _idx_rows(edge_index[0])
    didxp = _idx_rows(edge_index[1])
    parts = _edge_stage(feats, sidxp, didxp)
    return jnp.sum(parts)
